# baseline (speedup 1.0000x reference)
import jax
import jax.numpy as jnp
from jax.experimental import pallas as pl
from jax.experimental.pallas import tpu as pltpu


def _fused_kernel(x_ref, w_ref, adj_ref, b_ref, out_ref, s_ref):
    @pl.when(pl.program_id(0) == 0)
    def _():
        s_ref[...] = jnp.dot(x_ref[...], w_ref[...],
                             preferred_element_type=jnp.float32)

    out_ref[...] = jnp.dot(adj_ref[...].astype(jnp.bfloat16),
                           s_ref[...].astype(jnp.bfloat16),
                           preferred_element_type=jnp.float32) + b_ref[...]


def kernel(input, adj, weight, bias):
    N, din = input.shape
    dout = weight.shape[1]

    tm = min(256, N)
    out = pl.pallas_call(
        _fused_kernel,
        grid=(pl.cdiv(N, tm),),
        in_specs=[
            pl.BlockSpec((N, din), lambda i: (0, 0)),
            pl.BlockSpec((din, dout), lambda i: (0, 0)),
            pl.BlockSpec((tm, N), lambda i: (i, 0)),
            pl.BlockSpec((1, dout), lambda i: (0, 0)),
        ],
        out_specs=pl.BlockSpec((tm, dout), lambda i: (i, 0)),
        out_shape=jax.ShapeDtypeStruct((N, dout), jnp.float32),
        scratch_shapes=[pltpu.VMEM((N, dout), jnp.float32)],
        compiler_params=pltpu.CompilerParams(
            dimension_semantics=("arbitrary",)),
    )(input, weight, adj, bias)
    return out


# speedup vs baseline: 1.0082x; 1.0082x over previous
import jax
import jax.numpy as jnp
from jax.experimental import pallas as pl
from jax.experimental.pallas import tpu as pltpu


def _gcn_kernel(x_ref, w_ref, adj_ref, b_ref, out_ref):
    t = jnp.dot(adj_ref[...], x_ref[...], preferred_element_type=jnp.float32)
    out_ref[...] = jnp.dot(t, w_ref[...],
                           preferred_element_type=jnp.float32) + b_ref[...]


def kernel(input, adj, weight, bias):
    N, din = input.shape
    dout = weight.shape[1]

    tm = min(256, N)
    out = pl.pallas_call(
        _gcn_kernel,
        grid=(pl.cdiv(N, tm),),
        in_specs=[
            pl.BlockSpec((N, din), lambda i: (0, 0)),
            pl.BlockSpec((din, dout), lambda i: (0, 0)),
            pl.BlockSpec((tm, N), lambda i: (i, 0)),
            pl.BlockSpec((1, dout), lambda i: (0, 0)),
        ],
        out_specs=pl.BlockSpec((tm, dout), lambda i: (i, 0)),
        out_shape=jax.ShapeDtypeStruct((N, dout), jnp.float32),
        compiler_params=pltpu.CompilerParams(
            dimension_semantics=("arbitrary",)),
    )(input, weight, adj, bias)
    return out
